# Initial kernel scaffold; baseline (speedup 1.0000x reference)
import jax
import jax.numpy as jnp
from jax.experimental import pallas as pl


def kernel(x):
    raise NotImplementedError("write your pallas kernel here")


# speedup vs baseline: 1.4557x; 1.4557x over previous
import jax
import jax.numpy as jnp
from jax.experimental import pallas as pl
from jax.experimental.pallas import tpu as pltpu

_VOCAB = 1000
_SEQ = 50
_BATCH = 1024
_BLK = 32
_NRATINGS = (_SEQ - 1) // 2


def _body(x_ref, o_ref):
    xb = x_ref[...].astype(jnp.float32)
    col = jax.lax.broadcasted_iota(jnp.int32, (_BLK, _SEQ), 1)
    mask = (col >= 2) & (col % 2 == 0)
    s = jnp.sum(jnp.where(mask, xb, 0.0), axis=1).astype(jnp.int32)
    q = s // _NRATINGS
    r = s - q * _NRATINGS
    half = _NRATINGS // 2
    inc = (r > half) | ((r == half) & ((q & 1) == 1))
    avg = q + inc.astype(jnp.int32)

    voc = jax.lax.broadcasted_iota(jnp.int32, (_BLK, _VOCAB), 1)
    onehot = (voc == avg[:, None]).astype(jnp.float32)

    o_ref[...] = jnp.zeros((_BLK, _SEQ, _VOCAB), jnp.float32)
    o_ref[:, _SEQ - 1 : _SEQ, :] = onehot[:, None, :]


def kernel(x):
    return pl.pallas_call(
        _body,
        grid=(_BATCH // _BLK,),
        in_specs=[pl.BlockSpec((_BLK, _SEQ), lambda i: (i, 0))],
        out_specs=pl.BlockSpec((_BLK, _SEQ, _VOCAB), lambda i: (i, 0, 0)),
        out_shape=jax.ShapeDtypeStruct((_BATCH, _SEQ, _VOCAB), jnp.float32),
        compiler_params=pltpu.CompilerParams(
            dimension_semantics=("parallel",),
        ),
    )(x)
